# Initial kernel scaffold; baseline (speedup 1.0000x reference)
import jax
import jax.numpy as jnp
from jax.experimental import pallas as pl


def kernel(x, old_W, new_W):
    raise NotImplementedError("write your pallas kernel here")


# speedup vs baseline: 3.5807x; 3.5807x over previous
import functools

import jax
import jax.numpy as jnp
from jax.experimental import pallas as pl
from jax.experimental.pallas import tpu as pltpu
from jax.experimental.pallas import tpu_sc as plsc

_NUM_OLD = 500000
_D = 64
_L = 16
_W = 128


def kernel(x, old_W, new_W):
    B = x.shape[0]
    mesh = plsc.VectorSubcoreMesh(core_axis_name="core", subcore_axis_name="subcore")

    @functools.partial(
        pl.kernel,
        out_type=jax.ShapeDtypeStruct((B, _D), jnp.float32),
        mesh=mesh,
        scratch_types=[
            pltpu.VMEM((_W,), jnp.int32),
            pltpu.VMEM((_W,), jnp.int32),
            pltpu.VMEM((_W, _D), jnp.float32),
        ],
        compiler_params=pltpu.CompilerParams(use_tc_tiling_on_sc=False),
    )
    def run(x_hbm, oldw_hbm, neww_hbm, o_hbm, so_ref, sn_ref, ra_ref):
        def body(i_vmem, o_vmem):
            @pl.loop(0, _W, step=_L)
            def _build(g):
                v = i_vmem.at[0][pl.ds(g, _L)]
                m = v < _NUM_OLD
                so_ref[pl.ds(g, _L)] = jnp.where(m, v, 0)
                sn_ref[pl.ds(g, _L)] = jnp.where(m, 0, v - _NUM_OLD)

            pltpu.sync_copy(oldw_hbm.at[so_ref], ra_ref)
            pltpu.sync_copy(neww_hbm.at[sn_ref], o_vmem)

            @pl.loop(0, _W, step=_L)
            def _merge(g):
                v = i_vmem.at[0][pl.ds(g, _L)]
                for j in range(_L):
                    @pl.when(v[j] < _NUM_OLD)
                    def _():
                        for c in range(_D // _L):
                            o_vmem[g + j, pl.ds(c * _L, _L)] = ra_ref[g + j, pl.ds(c * _L, _L)]

        pltpu.emit_pipeline(
            body,
            grid=(B // _W,),
            in_specs=[pl.BlockSpec((1, _W), index_map=lambda i: (0, i))],
            out_specs=[pl.BlockSpec((_W, _D), index_map=lambda i: (i, 0))],
            core_axis_name=("core", "subcore"),
            dimension_semantics=(pltpu.PARALLEL,),
        )(x_hbm, o_hbm)

    return run(x.reshape(1, B), old_W, new_W)


# speedup vs baseline: 3.5865x; 1.0016x over previous
import functools

import jax
import jax.numpy as jnp
from jax.experimental import pallas as pl
from jax.experimental.pallas import tpu as pltpu
from jax.experimental.pallas import tpu_sc as plsc

_NUM_OLD = 500000
_D = 64
_L = 16
_W = 128


def kernel(x, old_W, new_W):
    B = x.shape[0]
    mesh = plsc.VectorSubcoreMesh(core_axis_name="core", subcore_axis_name="subcore")

    @functools.partial(
        pl.kernel,
        out_type=jax.ShapeDtypeStruct((B, _D), jnp.float32),
        mesh=mesh,
        scratch_types=[
            pltpu.VMEM((_W,), jnp.int32),
            pltpu.VMEM((_W,), jnp.int32),
            pltpu.VMEM((_W, _D), jnp.float32),
        ],
        compiler_params=pltpu.CompilerParams(use_tc_tiling_on_sc=False),
    )
    def run(x_hbm, oldw_hbm, neww_hbm, o_hbm, so_ref, sn_ref, ra_ref):
        def body(i_vmem, o_vmem):
            @pl.loop(0, _W, step=_L)
            def _build(g):
                v = i_vmem.at[0][pl.ds(g, _L)]
                m = v < _NUM_OLD
                so_ref[pl.ds(g, _L)] = jnp.where(m, v, 0)
                sn_ref[pl.ds(g, _L)] = jnp.where(m, 0, v - _NUM_OLD)

            pltpu.sync_copy(oldw_hbm.at[so_ref], ra_ref)
            pltpu.sync_copy(neww_hbm.at[sn_ref], o_vmem)

            @pl.loop(0, _W, step=_L)
            def _merge(g):
                v = i_vmem.at[0][pl.ds(g, _L)]
                for j in range(_L):
                    @pl.when(v[j] < _NUM_OLD)
                    def _():
                        for c in range(_D // _L):
                            o_vmem[g + j, pl.ds(c * _L, _L)] = ra_ref[g + j, pl.ds(c * _L, _L)]

        pltpu.emit_pipeline(
            body,
            grid=(B // _W,),
            in_specs=[pl.BlockSpec((1, _W), index_map=lambda i: (0, i))],
            out_specs=[pl.BlockSpec((_W, _D), index_map=lambda i: (i, 0))],
            core_axis_name=("core", "subcore"),
            dimension_semantics=(pltpu.PARALLEL,),
            trace_scopes=False,
        )(x_hbm, o_hbm)

    return run(x.reshape(1, B), old_W, new_W)


# speedup vs baseline: 3.5882x; 1.0005x over previous
import functools

import jax
import jax.numpy as jnp
from jax.experimental import pallas as pl
from jax.experimental.pallas import tpu as pltpu
from jax.experimental.pallas import tpu_sc as plsc

_NUM_OLD = 500000
_D = 64
_L = 16
_W = 128


def kernel(x, old_W, new_W):
    B = x.shape[0]
    mesh = plsc.VectorSubcoreMesh(core_axis_name="core", subcore_axis_name="subcore")

    @functools.partial(
        pl.kernel,
        out_type=jax.ShapeDtypeStruct((B, _D), jnp.float32),
        mesh=mesh,
        scratch_types=[
            pltpu.VMEM((_W,), jnp.int32),
            pltpu.VMEM((_W,), jnp.int32),
            pltpu.VMEM((_W, _D), jnp.float32),
        ],
        compiler_params=pltpu.CompilerParams(use_tc_tiling_on_sc=False),
    )
    def run(x_hbm, oldw_hbm, neww_hbm, o_hbm, so_ref, sn_ref, ra_ref):
        def body(i_vmem, o_vmem):
            @pl.loop(0, _W, step=_L)
            def _build(g):
                v = i_vmem.at[0][pl.ds(g, _L)]
                m = v < _NUM_OLD
                so_ref[pl.ds(g, _L)] = jnp.where(m, v, 0)
                sn_ref[pl.ds(g, _L)] = jnp.where(m, 0, v - _NUM_OLD)

            pltpu.sync_copy(oldw_hbm.at[so_ref], ra_ref)
            pltpu.sync_copy(neww_hbm.at[sn_ref], o_vmem)

            @pl.loop(0, _W, step=_L)
            def _merge(g):
                v = i_vmem.at[0][pl.ds(g, _L)]
                for j in range(_L):
                    @pl.when(v[j] < _NUM_OLD)
                    def _():
                        for c in range(_D // _L):
                            o_vmem[g + j, pl.ds(c * _L, _L)] = ra_ref[g + j, pl.ds(c * _L, _L)]

        wid = jax.lax.axis_index("subcore") * 2 + jax.lax.axis_index("core")
        nw = B // _W // 32
        pltpu.emit_pipeline(
            body,
            grid=(nw,),
            in_specs=[pl.BlockSpec((1, _W), index_map=lambda i: (0, wid * nw + i))],
            out_specs=[pl.BlockSpec((_W, _D), index_map=lambda i: (wid * nw + i, 0))],
            trace_scopes=False,
        )(x_hbm, o_hbm)

    return run(x.reshape(1, B), old_W, new_W)


# speedup vs baseline: 3.6618x; 1.0205x over previous
import functools

import jax
import jax.numpy as jnp
from jax.experimental import pallas as pl
from jax.experimental.pallas import tpu as pltpu
from jax.experimental.pallas import tpu_sc as plsc

_NUM_OLD = 500000
_D = 64
_L = 16
_W = 512
_C = 64


def kernel(x, old_W, new_W):
    B = x.shape[0]
    mesh = plsc.VectorSubcoreMesh(core_axis_name="core", subcore_axis_name="subcore")

    @functools.partial(
        pl.kernel,
        out_type=jax.ShapeDtypeStruct((B, _D), jnp.float32),
        mesh=mesh,
        scratch_types=[
            pltpu.VMEM((_W,), jnp.int32),
            pltpu.VMEM((_W,), jnp.int32),
            pltpu.VMEM((_W, _D), jnp.float32),
            pltpu.SemaphoreType.DMA,
        ],
        compiler_params=pltpu.CompilerParams(use_tc_tiling_on_sc=False),
    )
    def run(x_hbm, oldw_hbm, neww_hbm, o_hbm, so_ref, sn_ref, ra_ref, sem):
        def body(i_vmem, o_vmem):
            @pl.loop(0, _W, step=_L)
            def _build(g):
                v = i_vmem.at[0][pl.ds(g, _L)]
                m = v < _NUM_OLD
                so_ref[pl.ds(g, _L)] = jnp.where(m, v, 0)
                sn_ref[pl.ds(g, _L)] = jnp.where(m, 0, v - _NUM_OLD)

            copies = []
            for k in range(_W // _C):
                sl = pl.ds(k * _C, _C)
                copies.append(pltpu.async_copy(
                    oldw_hbm.at[so_ref.at[sl]], ra_ref.at[sl], sem))
                copies.append(pltpu.async_copy(
                    neww_hbm.at[sn_ref.at[sl]], o_vmem.at[sl], sem))
            for cp in copies:
                cp.wait()

            @pl.loop(0, _W, step=_L)
            def _merge(g):
                v = i_vmem.at[0][pl.ds(g, _L)]
                for j in range(_L):
                    @pl.when(v[j] < _NUM_OLD)
                    def _():
                        for c in range(_D // _L):
                            o_vmem[g + j, pl.ds(c * _L, _L)] = ra_ref[g + j, pl.ds(c * _L, _L)]

        pltpu.emit_pipeline(
            body,
            grid=(B // _W,),
            in_specs=[pl.BlockSpec((1, _W), index_map=lambda i: (0, i))],
            out_specs=[pl.BlockSpec((_W, _D), index_map=lambda i: (i, 0))],
            core_axis_name=("core", "subcore"),
            dimension_semantics=(pltpu.PARALLEL,),
            trace_scopes=False,
        )(x_hbm, o_hbm)

    return run(x.reshape(1, B), old_W, new_W)


# speedup vs baseline: 24.7340x; 6.7545x over previous
import functools

import jax
import jax.numpy as jnp
from jax.experimental import pallas as pl
from jax.experimental.pallas import tpu as pltpu
from jax.experimental.pallas import tpu_sc as plsc

_NUM_OLD = 500000
_D = 64
_L = 16
_NT = 32
_C = 128


def _iota():
    return jax.lax.broadcasted_iota(jnp.int32, (_L,), 0)


def kernel(x, old_W, new_W):
    B = x.shape[0]
    PT = B // _NT
    NG = PT // _L
    LIST = PT + _C + _L
    mesh = plsc.VectorSubcoreMesh(core_axis_name="core", subcore_axis_name="subcore")

    @functools.partial(
        pl.kernel,
        out_type=jax.ShapeDtypeStruct((B, _D), jnp.float32),
        mesh=mesh,
        scratch_types=[
            pltpu.VMEM((PT,), jnp.int32),
            pltpu.VMEM((LIST,), jnp.int32),
            pltpu.VMEM((LIST,), jnp.int32),
            pltpu.VMEM((_C,), jnp.int32),
            pltpu.VMEM((_C,), jnp.int32),
            pltpu.VMEM((_C, _D), jnp.float32),
            pltpu.SMEM((8,), jnp.int32),
            pltpu.SemaphoreType.DMA,
        ],
        compiler_params=pltpu.CompilerParams(
            use_tc_tiling_on_sc=False, needs_layout_passes=False),
    )
    def run(x_hbm, oldw_hbm, neww_hbm, o_hbm,
            idx_ref, po_ref, pn_ref, srow_ref, spos_ref, rows_ref, cnt_ref, sem):
        wid = jax.lax.axis_index("subcore") * 2 + jax.lax.axis_index("core")
        base = wid * PT

        pltpu.sync_copy(x_hbm.at[pl.ds(base, PT)], idx_ref)

        cnt_ref[0] = 0
        cnt_ref[1] = 0

        @pl.loop(0, NG)
        def _filter(g):
            v = idx_ref[pl.ds(g * _L, _L)]
            pos16 = _iota() + g * _L
            m = v < _NUM_OLD
            mi = m.astype(jnp.int32)
            incl = plsc.cumsum(mi)
            co = cnt_ref[0]
            plsc.store_scatter(po_ref, [incl - mi + co], pos16, mask=m)
            cnt_ref[0] = co + incl[_L - 1]
            ni = 1 - mi
            incl2 = plsc.cumsum(ni)
            cn = cnt_ref[1]
            plsc.store_scatter(pn_ref, [incl2 - ni + cn], pos16, mask=~m)
            cnt_ref[1] = cn + incl2[_L - 1]

        for s in range(2):
            lst = po_ref if s == 0 else pn_ref
            n = cnt_ref[s]

            @pl.when(n > 0)
            def _pad():
                last = plsc.load_gather(lst, [jnp.full((_L,), n - 1, jnp.int32)])
                for k in range(_C // _L):
                    plsc.store_scatter(lst, [_iota() + (n + k * _L)], last)

        def sweep(lst, table, row_off):
            def chunk(c, _):
                o = c * _C
                for k in range(_C // _L):
                    ii = _iota() + (o + k * _L)
                    pv = plsc.load_gather(lst, [ii])
                    xv = plsc.load_gather(idx_ref, [pv])
                    srow_ref[pl.ds(k * _L, _L)] = xv - row_off
                    spos_ref[pl.ds(k * _L, _L)] = pv + base
                pltpu.sync_copy(table.at[srow_ref], rows_ref)
                pltpu.sync_copy(rows_ref, o_hbm.at[spos_ref])
                return 0

            nch = (cnt_ref[0 if row_off == 0 else 1] + (_C - 1)) // _C
            jax.lax.fori_loop(0, nch, chunk, 0)

        sweep(po_ref, oldw_hbm, 0)
        plsc.subcore_barrier()
        sweep(pn_ref, neww_hbm, _NUM_OLD)

    return run(x, old_W, new_W)


# speedup vs baseline: 27.0165x; 1.0923x over previous
import functools

import jax
import jax.numpy as jnp
from jax.experimental import pallas as pl
from jax.experimental.pallas import tpu as pltpu
from jax.experimental.pallas import tpu_sc as plsc

_NUM_OLD = 500000
_D = 64
_L = 16
_NT = 32
_C = 128


def _iota():
    return jax.lax.broadcasted_iota(jnp.int32, (_L,), 0)


def kernel(x, old_W, new_W):
    B = x.shape[0]
    PT = B // _NT
    NG = PT // _L
    LIST = PT + _C + _L
    mesh = plsc.VectorSubcoreMesh(core_axis_name="core", subcore_axis_name="subcore")

    @functools.partial(
        pl.kernel,
        out_type=jax.ShapeDtypeStruct((B, _D), jnp.float32),
        mesh=mesh,
        scratch_types=[
            pltpu.VMEM((PT,), jnp.int32),
            pltpu.VMEM((LIST,), jnp.int32),
            pltpu.VMEM((LIST,), jnp.int32),
            pltpu.VMEM((2, _C), jnp.int32),
            pltpu.VMEM((2, _C), jnp.int32),
            pltpu.VMEM((2, _C, _D), jnp.float32),
            pltpu.SMEM((8,), jnp.int32),
            pltpu.SemaphoreType.DMA,
            pltpu.SemaphoreType.DMA,
            pltpu.SemaphoreType.DMA,
            pltpu.SemaphoreType.DMA,
        ],
        compiler_params=pltpu.CompilerParams(
            use_tc_tiling_on_sc=False, needs_layout_passes=False),
    )
    def run(x_hbm, oldw_hbm, neww_hbm, o_hbm,
            idx_ref, po_ref, pn_ref, srow_ref, spos_ref, rows_ref, cnt_ref,
            sg0, sg1, ss0, ss1):
        wid = jax.lax.axis_index("subcore") * 2 + jax.lax.axis_index("core")
        base = wid * PT
        sem_g = (sg0, sg1)
        sem_s = (ss0, ss1)

        pltpu.sync_copy(x_hbm.at[pl.ds(base, PT)], idx_ref)

        cnt_ref[0] = 0
        cnt_ref[1] = 0

        @pl.loop(0, NG)
        def _filter(g):
            v = idx_ref[pl.ds(g * _L, _L)]
            pos16 = _iota() + g * _L
            m = v < _NUM_OLD
            mi = m.astype(jnp.int32)
            incl = plsc.cumsum(mi)
            co = cnt_ref[0]
            plsc.store_scatter(po_ref, [incl - mi + co], pos16, mask=m)
            cnt_ref[0] = co + incl[_L - 1]
            ni = 1 - mi
            incl2 = plsc.cumsum(ni)
            cn = cnt_ref[1]
            plsc.store_scatter(pn_ref, [incl2 - ni + cn], pos16, mask=~m)
            cnt_ref[1] = cn + incl2[_L - 1]

        for s in range(2):
            lst = po_ref if s == 0 else pn_ref
            n = cnt_ref[s]

            @pl.when(n > 0)
            def _pad():
                last = plsc.load_gather(lst, [jnp.full((_L,), n - 1, jnp.int32)])
                for k in range(_C // _L):
                    plsc.store_scatter(lst, [_iota() + (n + k * _L)], last)

        def sweep(lst, table, row_off, which):
            def stage(c, par):
                o = c * _C
                for k in range(_C // _L):
                    ii = _iota() + (o + k * _L)
                    pv = plsc.load_gather(lst, [ii])
                    xv = plsc.load_gather(idx_ref, [pv])
                    srow_ref[par, pl.ds(k * _L, _L)] = xv - row_off
                    spos_ref[par, pl.ds(k * _L, _L)] = pv + base

            def g_copy(par):
                return pltpu.make_async_copy(
                    table.at[srow_ref.at[par]], rows_ref.at[par], sem_g[par])

            def s_copy(par):
                return pltpu.make_async_copy(
                    rows_ref.at[par], o_hbm.at[spos_ref.at[par]], sem_s[par])

            nch = (cnt_ref[which] + (_C - 1)) // _C

            @pl.when(nch > 0)
            def _go():
                stage(0, 0)
                g_copy(0).start()

                def body(c, _):
                    def step(par):
                        stage(c, par)
                        @pl.when(c >= 2)
                        def _():
                            s_copy(par).wait()
                        g_copy(par).start()
                        g_copy(1 - par).wait()
                        s_copy(1 - par).start()

                    @pl.when(c % 2 == 0)
                    def _():
                        step(0)

                    @pl.when(c % 2 == 1)
                    def _():
                        step(1)

                    return 0

                jax.lax.fori_loop(1, nch, body, 0)

                def fin(lp):
                    g_copy(lp).wait()
                    s_copy(lp).start()
                    @pl.when(nch >= 2)
                    def _():
                        s_copy(1 - lp).wait()
                    s_copy(lp).wait()

                lp = (nch - 1) % 2

                @pl.when(lp == 0)
                def _():
                    fin(0)

                @pl.when(lp == 1)
                def _():
                    fin(1)

        sweep(po_ref, oldw_hbm, 0, 0)
        plsc.subcore_barrier()
        sweep(pn_ref, neww_hbm, _NUM_OLD, 1)

    return run(x, old_W, new_W)


# speedup vs baseline: 27.0855x; 1.0026x over previous
import functools

import jax
import jax.numpy as jnp
from jax.experimental import pallas as pl
from jax.experimental.pallas import tpu as pltpu
from jax.experimental.pallas import tpu_sc as plsc

_NUM_OLD = 500000
_D = 64
_L = 16
_NT = 32
_C = 128


def _iota():
    return jax.lax.broadcasted_iota(jnp.int32, (_L,), 0)


def kernel(x, old_W, new_W):
    B = x.shape[0]
    PT = B // _NT
    NG = PT // _L
    LIST = PT + _C + _L
    mesh = plsc.VectorSubcoreMesh(core_axis_name="core", subcore_axis_name="subcore")

    @functools.partial(
        pl.kernel,
        out_type=jax.ShapeDtypeStruct((B, _D), jnp.float32),
        mesh=mesh,
        scratch_types=[
            pltpu.VMEM((PT,), jnp.int32),
            pltpu.VMEM((LIST,), jnp.int32),
            pltpu.VMEM((LIST,), jnp.int32),
            pltpu.VMEM((2, _C), jnp.int32),
            pltpu.VMEM((2, _C), jnp.int32),
            pltpu.VMEM((2, _C, _D), jnp.float32),
            pltpu.SMEM((8,), jnp.int32),
            pltpu.SemaphoreType.DMA,
            pltpu.SemaphoreType.DMA,
            pltpu.SemaphoreType.DMA,
            pltpu.SemaphoreType.DMA,
        ],
        compiler_params=pltpu.CompilerParams(
            use_tc_tiling_on_sc=False, needs_layout_passes=False),
    )
    def run(x_hbm, oldw_hbm, neww_hbm, o_hbm,
            idx_ref, po_ref, pn_ref, srow_ref, spos_ref, rows_ref, cnt_ref,
            sg0, sg1, ss0, ss1):
        wid = jax.lax.axis_index("subcore") * 2 + jax.lax.axis_index("core")
        base = wid * PT
        sem_g = (sg0, sg1)
        sem_s = (ss0, ss1)

        pltpu.sync_copy(x_hbm.at[pl.ds(base, PT)], idx_ref)

        cnt_ref[0] = 0
        cnt_ref[1] = 0

        @pl.loop(0, NG)
        def _filter(g):
            v = idx_ref[pl.ds(g * _L, _L)]
            pos16 = _iota() + g * _L
            m = v < _NUM_OLD
            mi = m.astype(jnp.int32)
            incl = plsc.cumsum(mi)
            co = cnt_ref[0]
            plsc.store_scatter(po_ref, [incl - mi + co], pos16, mask=m)
            cnt_ref[0] = co + incl[_L - 1]
            ni = 1 - mi
            incl2 = plsc.cumsum(ni)
            cn = cnt_ref[1]
            plsc.store_scatter(pn_ref, [incl2 - ni + cn], pos16, mask=~m)
            cnt_ref[1] = cn + incl2[_L - 1]

        for s in range(2):
            lst = po_ref if s == 0 else pn_ref
            n = cnt_ref[s]

            @pl.when(n > 0)
            def _pad():
                last = plsc.load_gather(lst, [jnp.full((_L,), n - 1, jnp.int32)])
                for k in range(_C // _L):
                    plsc.store_scatter(lst, [_iota() + (n + k * _L)], last)

        def sweep(lst, table, row_off, which):
            def stage(c, par):
                o = c * _C
                for k in range(_C // _L):
                    ii = _iota() + (o + k * _L)
                    pv = plsc.load_gather(lst, [ii])
                    xv = plsc.load_gather(idx_ref, [pv])
                    srow_ref[par, pl.ds(k * _L, _L)] = xv - row_off
                    spos_ref[par, pl.ds(k * _L, _L)] = pv + base

            def g_copy(par):
                return pltpu.make_async_copy(
                    table.at[srow_ref.at[par]], rows_ref.at[par], sem_g[par])

            def s_copy(par):
                return pltpu.make_async_copy(
                    rows_ref.at[par], o_hbm.at[spos_ref.at[par]], sem_s[par])

            nch = (cnt_ref[which] + (_C - 1)) // _C

            @pl.when(nch > 0)
            def _go():
                stage(0, 0)
                g_copy(0).start()

                def body(c, _):
                    def step(par):
                        @pl.when(c >= 2)
                        def _():
                            s_copy(par).wait()
                        stage(c, par)
                        g_copy(par).start()
                        g_copy(1 - par).wait()
                        s_copy(1 - par).start()

                    @pl.when(c % 2 == 0)
                    def _():
                        step(0)

                    @pl.when(c % 2 == 1)
                    def _():
                        step(1)

                    return 0

                jax.lax.fori_loop(1, nch, body, 0)

                def fin(lp):
                    g_copy(lp).wait()
                    s_copy(lp).start()
                    @pl.when(nch >= 2)
                    def _():
                        s_copy(1 - lp).wait()
                    s_copy(lp).wait()

                lp = (nch - 1) % 2

                @pl.when(lp == 0)
                def _():
                    fin(0)

                @pl.when(lp == 1)
                def _():
                    fin(1)

        sweep(po_ref, oldw_hbm, 0, 0)
        plsc.subcore_barrier()
        sweep(pn_ref, neww_hbm, _NUM_OLD, 1)

    return run(x, old_W, new_W)
